# baseline (speedup 1.0000x reference)
import functools

import jax
import jax.numpy as jnp
from jax import lax
from jax.experimental import pallas as pl
from jax.experimental.pallas import tpu as pltpu
from jax.experimental.pallas import tpu_sc as plsc

_NW = 32
_C = 128


def _make_sc_kernel(E, D):
    nch = E // _C
    base_per_w = nch // _NW
    extra = nch - base_per_w * _NW

    @functools.partial(
        pl.kernel,
        out_type=jax.ShapeDtypeStruct((nch, _C), jnp.float32),
        mesh=plsc.VectorSubcoreMesh(core_axis_name="c", subcore_axis_name="s"),
        compiler_params=pltpu.CompilerParams(use_tc_tiling_on_sc=False,
                                             needs_layout_passes=False),
        scratch_types=[
            pltpu.VMEM((_C,), jnp.int32),
            pltpu.VMEM((_C,), jnp.int32),
            pltpu.VMEM((_C, D), jnp.float32),
            pltpu.VMEM((_C, D), jnp.float32),
            pltpu.VMEM((_C,), jnp.float32),
            pltpu.SemaphoreType.DMA,
            pltpu.SemaphoreType.DMA,
        ],
    )
    def sc_kernel(x_hbm, src_hbm, dst_hbm, out_hbm,
                  idx_u, idx_v, urows, vrows, scores, semu, semv):
        wid = lax.axis_index("s") * 2 + lax.axis_index("c")
        cbase = wid * base_per_w + jnp.minimum(wid, extra)
        nmine = base_per_w + jnp.where(wid < extra, 1, 0)
        lane = lax.iota(jnp.int32, 16)

        def chunk_body(ci, carry):
            c = cbase + ci
            pltpu.sync_copy(src_hbm.at[c], idx_u)
            pltpu.sync_copy(dst_hbm.at[c], idx_v)
            cu = pltpu.async_copy(x_hbm.at[idx_u], urows, semu)
            cv = pltpu.async_copy(x_hbm.at[idx_v], vrows, semv)
            cu.wait()
            cv.wait()
            for g in range(_C // 16):
                rows = g * 16 + lane

                def dbody(d, acc):
                    cols = jnp.full((16,), 0, jnp.int32) + d
                    uu = plsc.load_gather(urows, [rows, cols])
                    vv = plsc.load_gather(vrows, [rows, cols])
                    return acc + uu * vv

                acc = lax.fori_loop(0, D, dbody,
                                    jnp.zeros((16,), jnp.float32), unroll=8)
                scores[pl.ds(g * 16, 16)] = acc
            pltpu.sync_copy(scores, out_hbm.at[c])
            return carry

        lax.fori_loop(0, nmine, chunk_body, 0)

    return sc_kernel


def kernel(x, edge_index):
    N, D = x.shape
    E = edge_index.shape[1]
    nch = E // _C
    src = edge_index[0].astype(jnp.int32).reshape(nch, _C)
    dst = edge_index[1].astype(jnp.int32).reshape(nch, _C)
    out = _make_sc_kernel(E, D)(x, src, dst)
    return out.reshape(E)


# speedup vs baseline: 2.3688x; 2.3688x over previous
import functools

import jax
import jax.numpy as jnp
from jax import lax
from jax.experimental import pallas as pl
from jax.experimental.pallas import tpu as pltpu
from jax.experimental.pallas import tpu_sc as plsc

_NW = 32
_C = 128


def _make_sc_kernel(nch_w, D):
    @functools.partial(
        pl.kernel,
        out_type=jax.ShapeDtypeStruct((_NW, nch_w, _C), jnp.float32),
        mesh=plsc.VectorSubcoreMesh(core_axis_name="c", subcore_axis_name="s"),
        compiler_params=pltpu.CompilerParams(use_tc_tiling_on_sc=False,
                                             needs_layout_passes=False),
        scratch_types=[
            pltpu.VMEM((nch_w, _C), jnp.int32),
            pltpu.VMEM((nch_w, _C), jnp.int32),
            pltpu.VMEM((_C, D), jnp.float32),
            pltpu.VMEM((_C, D), jnp.float32),
            pltpu.VMEM((nch_w, _C), jnp.float32),
            pltpu.SemaphoreType.DMA,
            pltpu.SemaphoreType.DMA,
        ],
    )
    def sc_kernel(x_hbm, src_hbm, dst_hbm, out_hbm,
                  idx_u, idx_v, urows, vrows, scores, semu, semv):
        wid = lax.axis_index("s") * 2 + lax.axis_index("c")
        pltpu.sync_copy(src_hbm.at[wid], idx_u)
        pltpu.sync_copy(dst_hbm.at[wid], idx_v)
        lane = lax.iota(jnp.int32, 16)

        def chunk_body(i, carry):
            cu = pltpu.async_copy(x_hbm.at[idx_u.at[i]], urows, semu)
            cv = pltpu.async_copy(x_hbm.at[idx_v.at[i]], vrows, semv)
            cu.wait()
            cv.wait()
            for g in range(_C // 16):
                rows = g * 16 + lane

                def dbody(t, acc):
                    cols = (lane + t) & (D - 1)
                    uu = plsc.load_gather(urows, [rows, cols])
                    vv = plsc.load_gather(vrows, [rows, cols])
                    return acc + uu * vv

                acc = lax.fori_loop(0, D, dbody,
                                    jnp.zeros((16,), jnp.float32), unroll=8)
                scores[i, pl.ds(g * 16, 16)] = acc
            return carry

        lax.fori_loop(0, nch_w, chunk_body, 0)
        pltpu.sync_copy(scores, out_hbm.at[wid])

    return sc_kernel


def kernel(x, edge_index):
    N, D = x.shape
    E = edge_index.shape[1]
    nch_w = -(-E // (_NW * _C))
    e_pad = _NW * nch_w * _C
    src = edge_index[0].astype(jnp.int32)
    dst = edge_index[1].astype(jnp.int32)
    pad = jnp.zeros((e_pad - E,), jnp.int32)
    src = jnp.concatenate([src, pad]).reshape(_NW, nch_w, _C)
    dst = jnp.concatenate([dst, pad]).reshape(_NW, nch_w, _C)
    out = _make_sc_kernel(nch_w, D)(x, src, dst)
    return out.reshape(e_pad)[:E]
